# Initial kernel scaffold; baseline (speedup 1.0000x reference)
import jax
import jax.numpy as jnp
from jax.experimental import pallas as pl


def kernel(hidden, router_weight, router_bias):
    raise NotImplementedError("write your pallas kernel here")


# speedup vs baseline: 1.0406x; 1.0406x over previous
import functools

import jax
import jax.numpy as jnp
from jax.experimental import pallas as pl
from jax.experimental.pallas import tpu as pltpu

CAP_FACTOR = 0.5
AUX_W = 0.01
EPS = 1e-9


def _router_body(h_ref, w_ref, b_ref, probs_ref, mask_ref, aux_ref, logits_sc,
                 *, num_chunks, chunk, k):
    c = pl.program_id(1)
    x = h_ref[0].astype(jnp.bfloat16).astype(jnp.float32)
    w = w_ref[...].astype(jnp.bfloat16).astype(jnp.float32)
    logit_chunk = jnp.sum(x * w, axis=-1) + b_ref[0, 0]
    logits_sc[c, :] = logit_chunk

    @pl.when(c == num_chunks - 1)
    def _finish():
        logits = logits_sc[...]
        ibits = jax.lax.bitcast_convert_type(logits, jnp.int32)
        skey = jnp.where(ibits < 0, ibits ^ jnp.int32(0x7FFFFFFF), ibits)
        ukey = jax.lax.bitcast_convert_type(skey, jnp.uint32) ^ jnp.uint32(0x80000000)

        def _bit_step(t, T):
            cand = T | jax.lax.shift_left(jnp.uint32(1), jnp.uint32(31 - t))
            cnt = jnp.sum((ukey >= cand).astype(jnp.int32))
            return jnp.where(cnt >= k, cand, T)

        T = jax.lax.fori_loop(0, 32, _bit_step, jnp.uint32(0))

        gt = ukey > T
        eq = ukey == T
        r = k - jnp.sum(gt.astype(jnp.int32))
        idx = (jax.lax.broadcasted_iota(jnp.int32, logits.shape, 0) * chunk
               + jax.lax.broadcasted_iota(jnp.int32, logits.shape, 1))

        def _lb_step(_, lo_hi):
            lo, hi = lo_hi
            mid = (lo + hi) // 2
            cnt = jnp.sum((eq & (idx < mid)).astype(jnp.int32))
            take_hi = cnt >= r
            return (jnp.where(take_hi, lo, mid + 1), jnp.where(take_hi, mid, hi))

        _, m = jax.lax.fori_loop(0, 13, _lb_step,
                                 (jnp.int32(0), jnp.int32(num_chunks * chunk)))

        sel = gt | (eq & (idx < m))
        mask_f = sel.astype(jnp.float32)
        probs = jax.nn.sigmoid(logits)
        bce = -(mask_f * jnp.log(probs + EPS)
                + (1.0 - mask_f) * jnp.log(1.0 - probs + EPS))
        probs_ref[0] = probs
        mask_ref[0] = mask_f
        aux = AUX_W * jnp.sum(bce) / (num_chunks * chunk)
        i = pl.program_id(0)
        aux_ref[pl.ds(i, 1), :] = jnp.full((1, 128), aux, jnp.float32)


@jax.jit
def kernel(hidden, router_weight, router_bias):
    b, s, d = hidden.shape
    chunk = 512
    num_chunks = s // chunk
    k = int(s * CAP_FACTOR)

    w2 = router_weight.reshape(1, d).astype(jnp.float32)
    b2 = jnp.asarray(router_bias, jnp.float32).reshape(1, 1)

    body = functools.partial(_router_body, num_chunks=num_chunks, chunk=chunk, k=k)
    probs3, mask3, aux2 = pl.pallas_call(
        body,
        grid=(b, num_chunks),
        in_specs=[
            pl.BlockSpec((1, chunk, d), lambda i, c: (i, c, 0)),
            pl.BlockSpec((1, d), lambda i, c: (0, 0)),
            pl.BlockSpec((1, 1), lambda i, c: (0, 0)),
        ],
        out_specs=[
            pl.BlockSpec((1, num_chunks, chunk), lambda i, c: (i, 0, 0)),
            pl.BlockSpec((1, num_chunks, chunk), lambda i, c: (i, 0, 0)),
            pl.BlockSpec((b, 128), lambda i, c: (0, 0)),
        ],
        out_shape=[
            jax.ShapeDtypeStruct((b, num_chunks, chunk), jnp.float32),
            jax.ShapeDtypeStruct((b, num_chunks, chunk), jnp.float32),
            jax.ShapeDtypeStruct((b, 128), jnp.float32),
        ],
        scratch_shapes=[pltpu.VMEM((num_chunks, chunk), jnp.float32)],
        compiler_params=pltpu.CompilerParams(
            dimension_semantics=("arbitrary", "arbitrary")),
    )(hidden, w2, b2)

    return (probs3.reshape(b, s), mask3.reshape(b, s), aux2[:, 0])


# speedup vs baseline: 1.1532x; 1.1082x over previous
import functools

import jax
import jax.numpy as jnp
from jax.experimental import pallas as pl
from jax.experimental.pallas import tpu as pltpu

CAP_FACTOR = 0.5
AUX_W = 0.01
EPS = 1e-9


def _router_body(h_ref, w_ref, b_ref, probs_ref, mask_ref, aux_ref, logits_sc,
                 *, num_chunks, chunk, k):
    c = pl.program_id(1)
    x = h_ref[0].astype(jnp.bfloat16).astype(jnp.float32)
    w = w_ref[...].astype(jnp.bfloat16).astype(jnp.float32)
    logit_chunk = jnp.sum(x * w, axis=-1) + b_ref[0, 0]
    logits_sc[c, :] = logit_chunk

    @pl.when(c == num_chunks - 1)
    def _finish():
        logits = logits_sc[...]
        ibits = jax.lax.bitcast_convert_type(logits, jnp.int32)
        skey = jnp.where(ibits < 0, ibits ^ jnp.int32(0x7FFFFFFF), ibits)
        ukey = jax.lax.bitcast_convert_type(skey, jnp.uint32) ^ jnp.uint32(0x80000000)

        def _bit_step(t, T):
            cand = T | jax.lax.shift_left(jnp.uint32(1), jnp.uint32(31 - t))
            cnt = jnp.sum((ukey >= cand).astype(jnp.int32))
            return jnp.where(cnt >= k, cand, T)

        T = jax.lax.fori_loop(0, 32, _bit_step, jnp.uint32(0))

        gt = ukey > T
        eq = ukey == T
        r = k - jnp.sum(gt.astype(jnp.int32))
        idx = (jax.lax.broadcasted_iota(jnp.int32, logits.shape, 0) * chunk
               + jax.lax.broadcasted_iota(jnp.int32, logits.shape, 1))

        def _lb_step(_, lo_hi):
            lo, hi = lo_hi
            mid = (lo + hi) // 2
            cnt = jnp.sum((eq & (idx < mid)).astype(jnp.int32))
            take_hi = cnt >= r
            return (jnp.where(take_hi, lo, mid + 1), jnp.where(take_hi, mid, hi))

        _, m = jax.lax.fori_loop(0, 13, _lb_step,
                                 (jnp.int32(0), jnp.int32(num_chunks * chunk)))

        sel = gt | (eq & (idx < m))
        mask_f = sel.astype(jnp.float32)
        probs = jax.nn.sigmoid(logits)
        bce = -(mask_f * jnp.log(probs + EPS)
                + (1.0 - mask_f) * jnp.log(1.0 - probs + EPS))
        probs_ref[0] = probs
        mask_ref[0] = mask_f
        aux = AUX_W * jnp.sum(bce) / (num_chunks * chunk)
        i = pl.program_id(0)
        aux_ref[pl.ds(i, 1), :] = jnp.full((1, 128), aux, jnp.float32)


@jax.jit
def kernel(hidden, router_weight, router_bias):
    b, s, d = hidden.shape
    chunk = 1024
    num_chunks = s // chunk
    k = int(s * CAP_FACTOR)

    w2 = router_weight.reshape(1, d).astype(jnp.float32)
    b2 = jnp.asarray(router_bias, jnp.float32).reshape(1, 1)

    body = functools.partial(_router_body, num_chunks=num_chunks, chunk=chunk, k=k)
    probs3, mask3, aux2 = pl.pallas_call(
        body,
        grid=(b, num_chunks),
        in_specs=[
            pl.BlockSpec((1, chunk, d), lambda i, c: (i, c, 0)),
            pl.BlockSpec((1, d), lambda i, c: (0, 0)),
            pl.BlockSpec((1, 1), lambda i, c: (0, 0)),
        ],
        out_specs=[
            pl.BlockSpec((1, num_chunks, chunk), lambda i, c: (i, 0, 0)),
            pl.BlockSpec((1, num_chunks, chunk), lambda i, c: (i, 0, 0)),
            pl.BlockSpec((b, 128), lambda i, c: (0, 0)),
        ],
        out_shape=[
            jax.ShapeDtypeStruct((b, num_chunks, chunk), jnp.float32),
            jax.ShapeDtypeStruct((b, num_chunks, chunk), jnp.float32),
            jax.ShapeDtypeStruct((b, 128), jnp.float32),
        ],
        scratch_shapes=[pltpu.VMEM((num_chunks, chunk), jnp.float32)],
        compiler_params=pltpu.CompilerParams(
            dimension_semantics=("arbitrary", "arbitrary")),
    )(hidden, w2, b2)

    return (probs3.reshape(b, s), mask3.reshape(b, s), aux2[:, 0])


# speedup vs baseline: 1.1640x; 1.0094x over previous
import functools

import jax
import jax.numpy as jnp
from jax.experimental import pallas as pl
from jax.experimental.pallas import tpu as pltpu

CAP_FACTOR = 0.5
AUX_W = 0.01
EPS = 1e-9


def _router_body(h_ref, w_ref, b_ref, probs_ref, mask_ref, aux_ref, logits_sc,
                 *, num_chunks, chunk, k):
    c = pl.program_id(1)
    x = h_ref[0].astype(jnp.bfloat16).astype(jnp.float32)
    w = w_ref[...].astype(jnp.bfloat16).astype(jnp.float32)
    logit_chunk = jnp.sum(x * w, axis=-1) + b_ref[0, 0]
    logits_sc[c, :] = logit_chunk

    @pl.when(c == num_chunks - 1)
    def _finish():
        logits = logits_sc[...]
        ibits = jax.lax.bitcast_convert_type(logits, jnp.int32)
        skey = jnp.where(ibits < 0, ibits ^ jnp.int32(0x7FFFFFFF), ibits)
        ukey = jax.lax.bitcast_convert_type(skey, jnp.uint32) ^ jnp.uint32(0x80000000)

        def _bit_step(t, T):
            cand = T | jax.lax.shift_left(jnp.uint32(1), jnp.uint32(31 - t))
            cnt = jnp.sum((ukey >= cand).astype(jnp.int32))
            return jnp.where(cnt >= k, cand, T)

        T = jax.lax.fori_loop(0, 32, _bit_step, jnp.uint32(0))

        gt = ukey > T
        eq = ukey == T
        r = k - jnp.sum(gt.astype(jnp.int32))
        idx = (jax.lax.broadcasted_iota(jnp.int32, logits.shape, 0) * chunk
               + jax.lax.broadcasted_iota(jnp.int32, logits.shape, 1))

        def _lb_step(_, lo_hi):
            lo, hi = lo_hi
            mid = (lo + hi) // 2
            cnt = jnp.sum((eq & (idx < mid)).astype(jnp.int32))
            take_hi = cnt >= r
            return (jnp.where(take_hi, lo, mid + 1), jnp.where(take_hi, mid, hi))

        _, m = jax.lax.fori_loop(0, 13, _lb_step,
                                 (jnp.int32(0), jnp.int32(num_chunks * chunk)))

        sel = gt | (eq & (idx < m))
        mask_f = sel.astype(jnp.float32)
        probs = jax.nn.sigmoid(logits)
        bce = -(mask_f * jnp.log(probs + EPS)
                + (1.0 - mask_f) * jnp.log(1.0 - probs + EPS))
        probs_ref[0] = probs
        mask_ref[0] = mask_f
        aux = AUX_W * jnp.sum(bce) / (num_chunks * chunk)
        i = pl.program_id(0)
        aux_ref[pl.ds(i, 1), :] = jnp.full((1, 128), aux, jnp.float32)


@jax.jit
def kernel(hidden, router_weight, router_bias):
    b, s, d = hidden.shape
    chunk = 2048
    num_chunks = s // chunk
    k = int(s * CAP_FACTOR)

    w2 = router_weight.reshape(1, d).astype(jnp.float32)
    b2 = jnp.asarray(router_bias, jnp.float32).reshape(1, 1)

    body = functools.partial(_router_body, num_chunks=num_chunks, chunk=chunk, k=k)
    probs3, mask3, aux2 = pl.pallas_call(
        body,
        grid=(b, num_chunks),
        in_specs=[
            pl.BlockSpec((1, chunk, d), lambda i, c: (i, c, 0)),
            pl.BlockSpec((1, d), lambda i, c: (0, 0)),
            pl.BlockSpec((1, 1), lambda i, c: (0, 0)),
        ],
        out_specs=[
            pl.BlockSpec((1, num_chunks, chunk), lambda i, c: (i, 0, 0)),
            pl.BlockSpec((1, num_chunks, chunk), lambda i, c: (i, 0, 0)),
            pl.BlockSpec((b, 128), lambda i, c: (0, 0)),
        ],
        out_shape=[
            jax.ShapeDtypeStruct((b, num_chunks, chunk), jnp.float32),
            jax.ShapeDtypeStruct((b, num_chunks, chunk), jnp.float32),
            jax.ShapeDtypeStruct((b, 128), jnp.float32),
        ],
        scratch_shapes=[pltpu.VMEM((num_chunks, chunk), jnp.float32)],
        compiler_params=pltpu.CompilerParams(
            dimension_semantics=("arbitrary", "arbitrary")),
    )(hidden, w2, b2)

    return (probs3.reshape(b, s), mask3.reshape(b, s), aux2[:, 0])
